# baseline (speedup 1.0000x reference)
import functools

import jax
import jax.numpy as jnp
from jax import lax
from jax.experimental import pallas as pl
from jax.experimental.pallas import tpu as pltpu
from jax.experimental.pallas import tpu_sc as plsc

_NUM_CLASSES = 21
_HIGH_T = 0.9
_LOW_T = 0.3
_B, _N, _G = 16, 20000, 32
_BLK = 4096
_NP = 20480
_NB = _NP // _BLK


def _iou_block(anct_ref, gt_ref):
    ax1 = anct_ref[0:1, :]
    ay1 = anct_ref[1:2, :]
    ax2 = anct_ref[2:3, :]
    ay2 = anct_ref[3:4, :]
    g = gt_ref[0]
    gx1 = g[:, 0:1]
    gy1 = g[:, 1:2]
    gx2 = g[:, 2:3]
    gy2 = g[:, 3:4]
    area_g = (gx2 - gx1) * (gy2 - gy1)
    area_a = (ax2 - ax1) * (ay2 - ay1)
    wx = jnp.maximum(jnp.minimum(gx2, ax2) - jnp.maximum(gx1, ax1), 0.0)
    wy = jnp.maximum(jnp.minimum(gy2, ay2) - jnp.maximum(gy1, ay1), 0.0)
    inter = wx * wy
    return inter / ((area_g + area_a) - inter)


def _hpg_kernel(anct_ref, gt_ref, hpg_ref):
    j = pl.program_id(1)
    mq = _iou_block(anct_ref, gt_ref)
    part = jnp.max(mq, axis=1, keepdims=True)

    @pl.when(j == 0)
    def _():
        hpg_ref[0] = part

    @pl.when(j > 0)
    def _():
        hpg_ref[0] = jnp.maximum(hpg_ref[0], part)


def _main_kernel(anct_ref, gt_ref, gl_ref, hpg_ref, conf_ref, pld_ref,
                 neg_ref, posl_ref, locl_ref, cepl_ref):
    b = pl.program_id(0)
    j = pl.program_id(1)

    mq = _iou_block(anct_ref, gt_ref)
    mv = jnp.max(mq, axis=0, keepdims=True)
    giota = jax.lax.broadcasted_iota(jnp.int32, mq.shape, 0).astype(jnp.float32)
    am = jnp.min(jnp.where(mq == mv, giota, float(_G)), axis=0, keepdims=True)
    m = jnp.where(mv < _LOW_T, -1.0, am)
    m = jnp.where((mv >= _LOW_T) & (mv < _HIGH_T), -2.0, m)
    eq = (mq == hpg_ref[0]).astype(jnp.float32)
    restore = jnp.max(eq, axis=0, keepdims=True) > 0.0
    mi = jnp.where(restore, am, m)
    lane = jax.lax.broadcasted_iota(jnp.int32, (1, _BLK), 1)
    pad = (j * _BLK + lane) >= _N
    mi = jnp.where(pad, -1.0, mi)

    idx = jnp.maximum(mi, 0.0)
    soh = (giota == idx).astype(jnp.float32)
    m5 = jnp.dot(gl_ref[0], soh, preferred_element_type=jnp.float32)
    labm = m5[4:5, :]
    ml = jnp.where(mi < 0.0, 0.0, labm)
    pos = ml > 0.0
    posf = pos.astype(jnp.float32)

    ax1 = anct_ref[0:1, :]
    ay1 = anct_ref[1:2, :]
    ax2 = anct_ref[2:3, :]
    ay2 = anct_ref[3:4, :]
    aw = ax2 - ax1
    ah = ay2 - ay1
    acx = (ax1 + ax2) * 0.5
    acy = (ay1 + ay2) * 0.5
    mx1 = m5[0:1, :]
    my1 = m5[1:2, :]
    mx2 = m5[2:3, :]
    my2 = m5[3:4, :]
    mw = mx2 - mx1
    mh = my2 - my1
    mcx = (mx1 + mx2) * 0.5
    mcy = (my1 + my2) * 0.5
    gcx = (mcx - acx) / (0.1 * aw)
    gcy = (mcy - acy) / (0.1 * ah)
    gw = jnp.log(mw / aw) / 0.2
    gh = jnp.log(mh / ah) / 0.2

    pld = pld_ref[0]

    def _sl1(d):
        ad = jnp.abs(d)
        return jnp.where(ad < 1.0, 0.5 * d * d, ad - 0.5)

    lrow = (_sl1(pld[0:1, :] - gcx) + _sl1(pld[1:2, :] - gcy)
            + _sl1(pld[2:3, :] - gw) + _sl1(pld[3:4, :] - gh)) * posf

    x = conf_ref[0]
    e = jnp.exp(x)
    s = jnp.sum(e, axis=0, keepdims=True)
    cio = jax.lax.broadcasted_iota(jnp.int32, x.shape, 0).astype(jnp.float32)
    ohc = (cio == ml).astype(jnp.float32)
    xl = jnp.sum(x * ohc, axis=0, keepdims=True)
    ce = jnp.log(s) - xl

    @pl.when(j == 0)
    def _():
        posl_ref[0] = posf

    @pl.when(j > 0)
    def _():
        posl_ref[0] += posf

    @pl.when((b == 0) & (j == 0))
    def _():
        locl_ref[0] = lrow
        cepl_ref[0] = ce * posf

    @pl.when((b > 0) | (j > 0))
    def _():
        locl_ref[0] += lrow
        cepl_ref[0] += ce * posf

    neg_ref[0, 0] = jnp.where(pos | pad, 0.0, ce)


_SCV = _NP // 16
_SWU = 8


def _sc_mine_kernel(neg_hbm, posl_hbm, locl_hbm, cepl_hbm,
                    out1_hbm, out2_hbm,
                    negv, poslv, c0, s0, c1, s1, c2, s2, stage):
    cid = lax.axis_index("c")
    sid = lax.axis_index("s")
    wid = sid * 2 + cid
    lanes_f = lax.iota(jnp.int32, 16).astype(jnp.float32)
    ones = jnp.full((16,), 1.0, jnp.float32)
    zeros = jnp.full((16,), 0.0, jnp.float32)

    def _vsum(ref, nv):
        def body(i, acc):
            return acc + ref[pl.ds(pl.multiple_of(i * 16, 16), 16)]
        return jnp.sum(lax.fori_loop(0, nv, body, zeros))

    @pl.when(wid < _B)
    def _():
        pltpu.sync_copy(neg_hbm.at[wid], negv)
        pltpu.sync_copy(posl_hbm.at[wid], poslv)
        np_b = _vsum(poslv, _BLK // 16)
        kk = jnp.minimum(3.0 * np_b, float(_N))

        def zero_hists(cref, sref, nv):
            def body(i, _):
                off = pl.ds(pl.multiple_of(i * 16, 16), 16)
                cref[off] = zeros
                sref[off] = zeros
                return 0
            lax.fori_loop(0, nv, body, 0)

        def sweep(level, p0, p01):
            def body(i, _):
                for u in range(_SWU):
                    off = pl.ds(pl.multiple_of((i * _SWU + u) * 16, 16), 16)
                    v = negv[off]
                    bits = plsc.bitcast(v, jnp.int32)
                    if level == 0:
                        idx = lax.shift_right_logical(bits, 20)
                        plsc.addupdate_scatter(c0, [idx], ones)
                        plsc.addupdate_scatter(s0, [idx], v)
                    elif level == 1:
                        msk = lax.shift_right_logical(bits, 20) == p0
                        idx = lax.shift_right_logical(bits, 10) & 1023
                        plsc.addupdate_scatter(c1, [idx], ones, mask=msk)
                        plsc.addupdate_scatter(s1, [idx], v, mask=msk)
                    else:
                        msk = lax.shift_right_logical(bits, 10) == p01
                        idx = bits & 1023
                        plsc.addupdate_scatter(c2, [idx], ones, mask=msk)
                        plsc.addupdate_scatter(s2, [idx], v, mask=msk)
                return 0
            lax.fori_loop(0, _SCV // _SWU, body, 0)

        def scan(cref, sref, nv, k_lvl):
            def body(t, carry):
                cnt_hi, sum_hi, jstar, cab, sab = carry
                i = nv - 1 - t
                off = pl.ds(pl.multiple_of(i * 16, 16), 16)
                c = cref[off]
                s = sref[off]
                pc = plsc.cumsum(c)
                ps = plsc.cumsum(s)
                tcs = jnp.sum(c)
                tss = jnp.sum(s)
                above = cnt_hi + (tcs - pc)
                sel = ((above < k_lvl) & (above + c >= k_lvl)).astype(jnp.float32)
                jstar = jstar + jnp.sum(sel * (i.astype(jnp.float32) * 16.0 + lanes_f))
                cab = cab + jnp.sum(sel * above)
                sab = sab + jnp.sum(sel * (sum_hi + (tss - ps)))
                return cnt_hi + tcs, sum_hi + tss, jstar, cab, sab
            return lax.fori_loop(0, nv, body, (0.0, 0.0, 0.0, 0.0, 0.0))

        zero_hists(c0, s0, 2048 // 16)
        zero_hists(c1, s1, 1024 // 16)
        zero_hists(c2, s2, 1024 // 16)

        sweep(0, 0, 0)
        _, _, j0, cab0, sab0 = scan(c0, s0, 2048 // 16, kk)
        j0i = j0.astype(jnp.int32)

        sweep(1, j0i, 0)
        _, _, j1, cab1, sab1 = scan(c1, s1, 1024 // 16, kk - cab0)
        j1i = j1.astype(jnp.int32)
        p01 = (j0i << 10) | j1i

        sweep(2, 0, p01)
        _, _, j2, cab2, sab2 = scan(c2, s2, 1024 // 16, kk - cab0 - cab1)
        j2i = j2.astype(jnp.int32)

        tbits = (p01 << 10) | j2i
        tstar = jnp.max(plsc.bitcast(jnp.full((16,), 1, jnp.int32) * tbits,
                                     jnp.float32))
        cab = cab0 + cab1 + cab2
        sab = sab0 + sab1 + sab2
        hard_b = jnp.where(kk > 0.0, sab + (kk - cab) * tstar, 0.0)
        stage[...] = jnp.where(lanes_f == 0.0, hard_b,
                               jnp.where(lanes_f == 1.0, np_b, 0.0))
        pltpu.sync_copy(stage, out1_hbm.at[wid])

    @pl.when(wid == _B)
    def _():
        pltpu.sync_copy(locl_hbm.at[0], poslv)
        def body(i, acc):
            return acc + poslv[pl.ds(pl.multiple_of(i * 16, 16), 16)]
        stage[...] = lax.fori_loop(0, _BLK // 16, body, zeros)
        pltpu.sync_copy(stage, out2_hbm.at[0])

    @pl.when(wid == _B + 1)
    def _():
        pltpu.sync_copy(cepl_hbm.at[0], poslv)
        def body(i, acc):
            return acc + poslv[pl.ds(pl.multiple_of(i * 16, 16), 16)]
        stage[...] = lax.fori_loop(0, _BLK // 16, body, zeros)
        pltpu.sync_copy(stage, out2_hbm.at[1])


_sc_mine = pl.kernel(
    _sc_mine_kernel,
    mesh=plsc.VectorSubcoreMesh(core_axis_name="c", subcore_axis_name="s"),
    compiler_params=pltpu.CompilerParams(needs_layout_passes=False),
    out_type=[
        jax.ShapeDtypeStruct((_B, 16), jnp.float32),
        jax.ShapeDtypeStruct((2, 16), jnp.float32),
    ],
    scratch_types=[
        pltpu.VMEM((_NP,), jnp.float32),
        pltpu.VMEM((_BLK,), jnp.float32),
        pltpu.VMEM((2048,), jnp.float32),
        pltpu.VMEM((2048,), jnp.float32),
        pltpu.VMEM((1024,), jnp.float32),
        pltpu.VMEM((1024,), jnp.float32),
        pltpu.VMEM((1024,), jnp.float32),
        pltpu.VMEM((1024,), jnp.float32),
        pltpu.VMEM((16,), jnp.float32),
    ],
)


def kernel(preds_loc_delta, preds_conf, anchors, gt_boxes, gt_labels):
    anchors_xyxy = jnp.concatenate(
        [anchors[:, :2], anchors[:, :2] + anchors[:, 2:]], axis=1)
    gt_xyxy = jnp.concatenate(
        [gt_boxes[..., :2], gt_boxes[..., :2] + gt_boxes[..., 2:]], axis=-1)
    anct = jnp.zeros((4, _NP), jnp.float32).at[:, :_N].set(anchors_xyxy.T)
    gl = jnp.concatenate(
        [gt_xyxy.transpose(0, 2, 1),
         gt_labels.astype(jnp.float32)[:, None, :]], axis=1)
    conf_t = jnp.zeros((_B, _NUM_CLASSES, _NP), jnp.float32).at[:, :, :_N].set(
        preds_conf.transpose(0, 2, 1))
    pld_t = jnp.zeros((_B, 4, _NP), jnp.float32).at[:, :, :_N].set(
        preds_loc_delta.transpose(0, 2, 1))

    hpg = pl.pallas_call(
        _hpg_kernel,
        grid=(_B, _NB),
        in_specs=[
            pl.BlockSpec((4, _BLK), lambda b, j: (0, j)),
            pl.BlockSpec((1, _G, 4), lambda b, j: (b, 0, 0)),
        ],
        out_specs=pl.BlockSpec((1, _G, 1), lambda b, j: (b, 0, 0)),
        out_shape=jax.ShapeDtypeStruct((_B, _G, 1), jnp.float32),
    )(anct, gt_xyxy)

    neg, posl, locl, cepl = pl.pallas_call(
        _main_kernel,
        grid=(_B, _NB),
        in_specs=[
            pl.BlockSpec((4, _BLK), lambda b, j: (0, j)),
            pl.BlockSpec((1, _G, 4), lambda b, j: (b, 0, 0)),
            pl.BlockSpec((1, 5, _G), lambda b, j: (b, 0, 0)),
            pl.BlockSpec((1, _G, 1), lambda b, j: (b, 0, 0)),
            pl.BlockSpec((1, _NUM_CLASSES, _BLK), lambda b, j: (b, 0, j)),
            pl.BlockSpec((1, 4, _BLK), lambda b, j: (b, 0, j)),
        ],
        out_specs=[
            pl.BlockSpec((1, 1, 1, _BLK), lambda b, j: (b, j, 0, 0)),
            pl.BlockSpec((1, 1, _BLK), lambda b, j: (b, 0, 0)),
            pl.BlockSpec((1, 1, _BLK), lambda b, j: (0, 0, 0)),
            pl.BlockSpec((1, 1, _BLK), lambda b, j: (0, 0, 0)),
        ],
        out_shape=[
            jax.ShapeDtypeStruct((_B, _NB, 1, _BLK), jnp.float32),
            jax.ShapeDtypeStruct((_B, 1, _BLK), jnp.float32),
            jax.ShapeDtypeStruct((1, 1, _BLK), jnp.float32),
            jax.ShapeDtypeStruct((1, 1, _BLK), jnp.float32),
        ],
    )(anct, gt_xyxy, gl, hpg, conf_t, pld_t)

    out1, out2 = _sc_mine(neg.reshape(_B, _NP), posl.reshape(_B, _BLK),
                          locl.reshape(1, _BLK), cepl.reshape(1, _BLK))
    hard_tot = out1[:, 0].sum()
    np_tot = out1[:, 1].sum()
    locsum = out2[0].sum()
    cepsum = out2[1].sum()
    lloc = locsum / jnp.maximum(np_tot * 4.0, 1.0)
    lconf = (hard_tot + cepsum) / jnp.maximum(np_tot, 1.0)
    return lloc, lconf


# speedup vs baseline: 1.0187x; 1.0187x over previous
import functools

import jax
import jax.numpy as jnp
from jax import lax
from jax.experimental import pallas as pl
from jax.experimental.pallas import tpu as pltpu
from jax.experimental.pallas import tpu_sc as plsc

_NUM_CLASSES = 21
_HIGH_T = 0.9
_LOW_T = 0.3
_B, _N, _G = 16, 20000, 32
_BLK = 4096
_NP = 20480
_NB = _NP // _BLK


def _iou_block(anct_ref, gt_ref):
    ax1 = anct_ref[0:1, :]
    ay1 = anct_ref[1:2, :]
    ax2 = anct_ref[2:3, :]
    ay2 = anct_ref[3:4, :]
    g = gt_ref[0]
    gx1 = g[:, 0:1]
    gy1 = g[:, 1:2]
    gx2 = g[:, 2:3]
    gy2 = g[:, 3:4]
    area_g = (gx2 - gx1) * (gy2 - gy1)
    area_a = (ax2 - ax1) * (ay2 - ay1)
    wx = jnp.maximum(jnp.minimum(gx2, ax2) - jnp.maximum(gx1, ax1), 0.0)
    wy = jnp.maximum(jnp.minimum(gy2, ay2) - jnp.maximum(gy1, ay1), 0.0)
    inter = wx * wy
    return inter / ((area_g + area_a) - inter)


def _hpg_kernel(anct_ref, gt_ref, hpg_ref):
    j = pl.program_id(1)
    mq = _iou_block(anct_ref, gt_ref)
    part = jnp.max(mq, axis=1, keepdims=True)

    @pl.when(j == 0)
    def _():
        hpg_ref[0] = part

    @pl.when(j > 0)
    def _():
        hpg_ref[0] = jnp.maximum(hpg_ref[0], part)


def _main_kernel(anct_ref, gt_ref, gl_ref, hpg_ref, conf_ref, pld_ref,
                 neg_ref, posl_ref, locl_ref, cepl_ref):
    b = pl.program_id(0)
    j = pl.program_id(1)

    mq = _iou_block(anct_ref, gt_ref)
    mv = jnp.max(mq, axis=0, keepdims=True)
    giota = jax.lax.broadcasted_iota(jnp.int32, mq.shape, 0).astype(jnp.float32)
    am = jnp.min(jnp.where(mq == mv, giota, float(_G)), axis=0, keepdims=True)
    m = jnp.where(mv < _LOW_T, -1.0, am)
    m = jnp.where((mv >= _LOW_T) & (mv < _HIGH_T), -2.0, m)
    eq = (mq == hpg_ref[0]).astype(jnp.float32)
    restore = jnp.max(eq, axis=0, keepdims=True) > 0.0
    mi = jnp.where(restore, am, m)
    lane = jax.lax.broadcasted_iota(jnp.int32, (1, _BLK), 1)
    pad = (j * _BLK + lane) >= _N
    mi = jnp.where(pad, -1.0, mi)

    idx = jnp.maximum(mi, 0.0)
    soh = (giota == idx).astype(jnp.float32)
    m5 = jnp.dot(gl_ref[0], soh, preferred_element_type=jnp.float32)
    labm = m5[4:5, :]
    ml = jnp.where(mi < 0.0, 0.0, labm)
    pos = ml > 0.0
    posf = pos.astype(jnp.float32)

    ax1 = anct_ref[0:1, :]
    ay1 = anct_ref[1:2, :]
    ax2 = anct_ref[2:3, :]
    ay2 = anct_ref[3:4, :]
    aw = ax2 - ax1
    ah = ay2 - ay1
    acx = (ax1 + ax2) * 0.5
    acy = (ay1 + ay2) * 0.5
    mx1 = m5[0:1, :]
    my1 = m5[1:2, :]
    mx2 = m5[2:3, :]
    my2 = m5[3:4, :]
    mw = mx2 - mx1
    mh = my2 - my1
    mcx = (mx1 + mx2) * 0.5
    mcy = (my1 + my2) * 0.5
    gcx = (mcx - acx) / (0.1 * aw)
    gcy = (mcy - acy) / (0.1 * ah)
    gw = jnp.log(mw / aw) / 0.2
    gh = jnp.log(mh / ah) / 0.2

    pld = pld_ref[0]

    def _sl1(d):
        ad = jnp.abs(d)
        return jnp.where(ad < 1.0, 0.5 * d * d, ad - 0.5)

    lrow = (_sl1(pld[0:1, :] - gcx) + _sl1(pld[1:2, :] - gcy)
            + _sl1(pld[2:3, :] - gw) + _sl1(pld[3:4, :] - gh)) * posf

    x = conf_ref[0]
    e = jnp.exp(x)
    s = jnp.sum(e, axis=0, keepdims=True)
    cio = jax.lax.broadcasted_iota(jnp.int32, x.shape, 0).astype(jnp.float32)
    ohc = (cio == ml).astype(jnp.float32)
    xl = jnp.sum(x * ohc, axis=0, keepdims=True)
    ce = jnp.log(s) - xl

    @pl.when(j == 0)
    def _():
        posl_ref[0] = posf

    @pl.when(j > 0)
    def _():
        posl_ref[0] += posf

    @pl.when((b == 0) & (j == 0))
    def _():
        locl_ref[0] = lrow
        cepl_ref[0] = ce * posf

    @pl.when((b > 0) | (j > 0))
    def _():
        locl_ref[0] += lrow
        cepl_ref[0] += ce * posf

    neg_ref[0, 0] = jnp.where(pos | pad, 0.0, ce)


_SCV = _NP // 16
_SWU = 8


def _sc_mine_kernel(neg_hbm, posl_hbm, locl_hbm, cepl_hbm,
                    out1_hbm, out2_hbm,
                    negv, poslv, c0, c1, c2, stage):
    cid = lax.axis_index("c")
    sid = lax.axis_index("s")
    wid = sid * 2 + cid
    lanes_f = lax.iota(jnp.int32, 16).astype(jnp.float32)
    ones = jnp.full((16,), 1.0, jnp.float32)
    zeros = jnp.full((16,), 0.0, jnp.float32)

    def _vsum(ref, nv):
        def body(i, acc):
            return acc + ref[pl.ds(pl.multiple_of(i * 16, 16), 16)]
        return jnp.sum(lax.fori_loop(0, nv, body, zeros))

    @pl.when(wid < _B)
    def _():
        pltpu.sync_copy(neg_hbm.at[wid], negv)
        pltpu.sync_copy(posl_hbm.at[wid], poslv)
        np_b = _vsum(poslv, _BLK // 16)
        kk = jnp.minimum(3.0 * np_b, float(_N))

        def zero_hist(cref, nv):
            def body(i, _):
                cref[pl.ds(pl.multiple_of(i * 16, 16), 16)] = zeros
                return 0
            lax.fori_loop(0, nv, body, 0)

        def sweep(level, p0, p01):
            def body(i, _):
                for u in range(_SWU):
                    off = pl.ds(pl.multiple_of((i * _SWU + u) * 16, 16), 16)
                    bits = plsc.bitcast(negv[off], jnp.int32)
                    if level == 0:
                        idx = lax.shift_right_logical(bits, 20)
                        plsc.addupdate_scatter(c0, [idx], ones)
                    elif level == 1:
                        msk = lax.shift_right_logical(bits, 20) == p0
                        idx = lax.shift_right_logical(bits, 10) & 1023
                        plsc.addupdate_scatter(c1, [idx], ones, mask=msk)
                    else:
                        msk = lax.shift_right_logical(bits, 10) == p01
                        idx = bits & 1023
                        plsc.addupdate_scatter(c2, [idx], ones, mask=msk)
                return 0
            lax.fori_loop(0, _SCV // _SWU, body, 0)

        def scan(cref, nv, k_lvl):
            def body(t, carry):
                cnt_hi, jstar, cab = carry
                i = nv - 1 - t
                c = cref[pl.ds(pl.multiple_of(i * 16, 16), 16)]
                pc = plsc.cumsum(c)
                tcs = jnp.sum(c)
                above = cnt_hi + (tcs - pc)
                sel = ((above < k_lvl) & (above + c >= k_lvl)).astype(jnp.float32)
                jstar = jstar + jnp.sum(sel * (i.astype(jnp.float32) * 16.0 + lanes_f))
                cab = cab + jnp.sum(sel * above)
                return cnt_hi + tcs, jstar, cab
            return lax.fori_loop(0, nv, body, (0.0, 0.0, 0.0))

        zero_hist(c0, 2048 // 16)
        zero_hist(c1, 1024 // 16)
        zero_hist(c2, 1024 // 16)

        sweep(0, 0, 0)
        _, j0, cab0 = scan(c0, 2048 // 16, kk)
        j0i = j0.astype(jnp.int32)

        sweep(1, j0i, 0)
        _, j1, cab1 = scan(c1, 1024 // 16, kk - cab0)
        j1i = j1.astype(jnp.int32)
        p01 = (j0i << 10) | j1i

        sweep(2, 0, p01)
        _, j2, _ = scan(c2, 1024 // 16, kk - cab0 - cab1)
        j2i = j2.astype(jnp.int32)

        tbits = (p01 << 10) | j2i
        tsv = plsc.bitcast(jnp.full((16,), 1, jnp.int32) * tbits, jnp.float32)
        tstar = jnp.max(tsv)
        def gt_body(i, carry):
            cacc, sacc = carry
            for u in range(_SWU):
                off = pl.ds(pl.multiple_of((i * _SWU + u) * 16, 16), 16)
                v = negv[off]
                m = v > tsv
                cacc = cacc + jnp.where(m, 1.0, 0.0)
                sacc = sacc + jnp.where(m, v, 0.0)
            return cacc, sacc
        cacc, sacc = lax.fori_loop(0, _SCV // _SWU, gt_body, (zeros, zeros))
        cab = jnp.sum(cacc)
        sab = jnp.sum(sacc)
        hard_b = jnp.where(kk > 0.0, sab + (kk - cab) * tstar, 0.0)
        stage[...] = jnp.where(lanes_f == 0.0, hard_b,
                               jnp.where(lanes_f == 1.0, np_b, 0.0))
        pltpu.sync_copy(stage, out1_hbm.at[wid])

    @pl.when(wid == _B)
    def _():
        pltpu.sync_copy(locl_hbm.at[0], poslv)
        def body(i, acc):
            return acc + poslv[pl.ds(pl.multiple_of(i * 16, 16), 16)]
        stage[...] = lax.fori_loop(0, _BLK // 16, body, zeros)
        pltpu.sync_copy(stage, out2_hbm.at[0])

    @pl.when(wid == _B + 1)
    def _():
        pltpu.sync_copy(cepl_hbm.at[0], poslv)
        def body(i, acc):
            return acc + poslv[pl.ds(pl.multiple_of(i * 16, 16), 16)]
        stage[...] = lax.fori_loop(0, _BLK // 16, body, zeros)
        pltpu.sync_copy(stage, out2_hbm.at[1])


_sc_mine = pl.kernel(
    _sc_mine_kernel,
    mesh=plsc.VectorSubcoreMesh(core_axis_name="c", subcore_axis_name="s"),
    compiler_params=pltpu.CompilerParams(needs_layout_passes=False),
    out_type=[
        jax.ShapeDtypeStruct((_B, 16), jnp.float32),
        jax.ShapeDtypeStruct((2, 16), jnp.float32),
    ],
    scratch_types=[
        pltpu.VMEM((_NP,), jnp.float32),
        pltpu.VMEM((_BLK,), jnp.float32),
        pltpu.VMEM((2048,), jnp.float32),
        pltpu.VMEM((1024,), jnp.float32),
        pltpu.VMEM((1024,), jnp.float32),
        pltpu.VMEM((16,), jnp.float32),
    ],
)


def kernel(preds_loc_delta, preds_conf, anchors, gt_boxes, gt_labels):
    anchors_xyxy = jnp.concatenate(
        [anchors[:, :2], anchors[:, :2] + anchors[:, 2:]], axis=1)
    gt_xyxy = jnp.concatenate(
        [gt_boxes[..., :2], gt_boxes[..., :2] + gt_boxes[..., 2:]], axis=-1)
    anct = jnp.zeros((4, _NP), jnp.float32).at[:, :_N].set(anchors_xyxy.T)
    gl = jnp.concatenate(
        [gt_xyxy.transpose(0, 2, 1),
         gt_labels.astype(jnp.float32)[:, None, :]], axis=1)
    conf_t = jnp.zeros((_B, _NUM_CLASSES, _NP), jnp.float32).at[:, :, :_N].set(
        preds_conf.transpose(0, 2, 1))
    pld_t = jnp.zeros((_B, 4, _NP), jnp.float32).at[:, :, :_N].set(
        preds_loc_delta.transpose(0, 2, 1))

    hpg = pl.pallas_call(
        _hpg_kernel,
        grid=(_B, _NB),
        in_specs=[
            pl.BlockSpec((4, _BLK), lambda b, j: (0, j)),
            pl.BlockSpec((1, _G, 4), lambda b, j: (b, 0, 0)),
        ],
        out_specs=pl.BlockSpec((1, _G, 1), lambda b, j: (b, 0, 0)),
        out_shape=jax.ShapeDtypeStruct((_B, _G, 1), jnp.float32),
    )(anct, gt_xyxy)

    neg, posl, locl, cepl = pl.pallas_call(
        _main_kernel,
        grid=(_B, _NB),
        in_specs=[
            pl.BlockSpec((4, _BLK), lambda b, j: (0, j)),
            pl.BlockSpec((1, _G, 4), lambda b, j: (b, 0, 0)),
            pl.BlockSpec((1, 5, _G), lambda b, j: (b, 0, 0)),
            pl.BlockSpec((1, _G, 1), lambda b, j: (b, 0, 0)),
            pl.BlockSpec((1, _NUM_CLASSES, _BLK), lambda b, j: (b, 0, j)),
            pl.BlockSpec((1, 4, _BLK), lambda b, j: (b, 0, j)),
        ],
        out_specs=[
            pl.BlockSpec((1, 1, 1, _BLK), lambda b, j: (b, j, 0, 0)),
            pl.BlockSpec((1, 1, _BLK), lambda b, j: (b, 0, 0)),
            pl.BlockSpec((1, 1, _BLK), lambda b, j: (0, 0, 0)),
            pl.BlockSpec((1, 1, _BLK), lambda b, j: (0, 0, 0)),
        ],
        out_shape=[
            jax.ShapeDtypeStruct((_B, _NB, 1, _BLK), jnp.float32),
            jax.ShapeDtypeStruct((_B, 1, _BLK), jnp.float32),
            jax.ShapeDtypeStruct((1, 1, _BLK), jnp.float32),
            jax.ShapeDtypeStruct((1, 1, _BLK), jnp.float32),
        ],
    )(anct, gt_xyxy, gl, hpg, conf_t, pld_t)

    out1, out2 = _sc_mine(neg.reshape(_B, _NP), posl.reshape(_B, _BLK),
                          locl.reshape(1, _BLK), cepl.reshape(1, _BLK))
    hard_tot = out1[:, 0].sum()
    np_tot = out1[:, 1].sum()
    locsum = out2[0].sum()
    cepsum = out2[1].sum()
    lloc = locsum / jnp.maximum(np_tot * 4.0, 1.0)
    lconf = (hard_tot + cepsum) / jnp.maximum(np_tot, 1.0)
    return lloc, lconf


# speedup vs baseline: 1.0524x; 1.0330x over previous
import functools

import jax
import jax.numpy as jnp
from jax import lax
from jax.experimental import pallas as pl
from jax.experimental.pallas import tpu as pltpu
from jax.experimental.pallas import tpu_sc as plsc

_NUM_CLASSES = 21
_HIGH_T = 0.9
_LOW_T = 0.3
_B, _N, _G = 16, 20000, 32
_BLK = 4096
_NP = 20480
_NB = _NP // _BLK


def _iou_block(anct_ref, gt_ref):
    ax1 = anct_ref[0:1, :]
    ay1 = anct_ref[1:2, :]
    ax2 = anct_ref[2:3, :]
    ay2 = anct_ref[3:4, :]
    g = gt_ref[0]
    gx1 = g[:, 0:1]
    gy1 = g[:, 1:2]
    gx2 = g[:, 2:3]
    gy2 = g[:, 3:4]
    area_g = (gx2 - gx1) * (gy2 - gy1)
    area_a = (ax2 - ax1) * (ay2 - ay1)
    wx = jnp.maximum(jnp.minimum(gx2, ax2) - jnp.maximum(gx1, ax1), 0.0)
    wy = jnp.maximum(jnp.minimum(gy2, ay2) - jnp.maximum(gy1, ay1), 0.0)
    inter = wx * wy
    return inter / ((area_g + area_a) - inter)


def _main_kernel(anct_ref, gt_ref, gl_ref, conf_ref, pld_ref,
                 neg_ref, posl_ref, locl_ref, cepl_ref,
                 mqs_ref, hpg_ref):
    b = pl.program_id(0)
    p = pl.program_id(1)
    j = pl.program_id(2)

    @pl.when(p == 0)
    def _():
        mq = _iou_block(anct_ref, gt_ref)
        mqs_ref[pl.ds(j, 1)] = mq[None]
        part = jnp.max(mq, axis=1, keepdims=True)

        @pl.when(j == 0)
        def _():
            hpg_ref[...] = part

        @pl.when(j > 0)
        def _():
            hpg_ref[...] = jnp.maximum(hpg_ref[...], part)

    @pl.when(p == 1)
    def _():
        _match_and_losses(anct_ref, gt_ref, gl_ref, conf_ref, pld_ref,
                          neg_ref, posl_ref, locl_ref, cepl_ref,
                          mqs_ref, hpg_ref, b, j)


def _match_and_losses(anct_ref, gt_ref, gl_ref, conf_ref, pld_ref,
                      neg_ref, posl_ref, locl_ref, cepl_ref,
                      mqs_ref, hpg_ref, b, j):
    mq = mqs_ref[pl.ds(j, 1)][0]
    mv = jnp.max(mq, axis=0, keepdims=True)
    giota = jax.lax.broadcasted_iota(jnp.int32, mq.shape, 0).astype(jnp.float32)
    am = jnp.min(jnp.where(mq == mv, giota, float(_G)), axis=0, keepdims=True)
    m = jnp.where(mv < _LOW_T, -1.0, am)
    m = jnp.where((mv >= _LOW_T) & (mv < _HIGH_T), -2.0, m)
    eq = (mq == hpg_ref[...]).astype(jnp.float32)
    restore = jnp.max(eq, axis=0, keepdims=True) > 0.0
    mi = jnp.where(restore, am, m)
    lane = jax.lax.broadcasted_iota(jnp.int32, (1, _BLK), 1)
    pad = (j * _BLK + lane) >= _N
    mi = jnp.where(pad, -1.0, mi)

    idx = jnp.maximum(mi, 0.0)
    soh = (giota == idx).astype(jnp.float32)
    m5 = jnp.dot(gl_ref[0], soh, preferred_element_type=jnp.float32)
    labm = m5[4:5, :]
    ml = jnp.where(mi < 0.0, 0.0, labm)
    pos = ml > 0.0
    posf = pos.astype(jnp.float32)

    ax1 = anct_ref[0:1, :]
    ay1 = anct_ref[1:2, :]
    ax2 = anct_ref[2:3, :]
    ay2 = anct_ref[3:4, :]
    aw = ax2 - ax1
    ah = ay2 - ay1
    acx = (ax1 + ax2) * 0.5
    acy = (ay1 + ay2) * 0.5
    mx1 = m5[0:1, :]
    my1 = m5[1:2, :]
    mx2 = m5[2:3, :]
    my2 = m5[3:4, :]
    mw = mx2 - mx1
    mh = my2 - my1
    mcx = (mx1 + mx2) * 0.5
    mcy = (my1 + my2) * 0.5
    gcx = (mcx - acx) / (0.1 * aw)
    gcy = (mcy - acy) / (0.1 * ah)
    gw = jnp.log(mw / aw) / 0.2
    gh = jnp.log(mh / ah) / 0.2

    pld = pld_ref[0]

    def _sl1(d):
        ad = jnp.abs(d)
        return jnp.where(ad < 1.0, 0.5 * d * d, ad - 0.5)

    lrow = (_sl1(pld[0:1, :] - gcx) + _sl1(pld[1:2, :] - gcy)
            + _sl1(pld[2:3, :] - gw) + _sl1(pld[3:4, :] - gh)) * posf

    x = conf_ref[0]
    e = jnp.exp(x)
    s = jnp.sum(e, axis=0, keepdims=True)
    cio = jax.lax.broadcasted_iota(jnp.int32, x.shape, 0).astype(jnp.float32)
    ohc = (cio == ml).astype(jnp.float32)
    xl = jnp.sum(x * ohc, axis=0, keepdims=True)
    ce = jnp.log(s) - xl

    @pl.when(j == 0)
    def _():
        posl_ref[0] = posf

    @pl.when(j > 0)
    def _():
        posl_ref[0] += posf

    @pl.when((b == 0) & (j == 0))
    def _():
        locl_ref[0] = lrow
        cepl_ref[0] = ce * posf

    @pl.when((b > 0) | (j > 0))
    def _():
        locl_ref[0] += lrow
        cepl_ref[0] += ce * posf

    neg_ref[0, 0] = jnp.where(pos | pad, 0.0, ce)


_SCV = _NP // 16
_SWU = 16


def _sc_mine_kernel(neg_hbm, posl_hbm, locl_hbm, cepl_hbm,
                    out1_hbm, out2_hbm,
                    negv, poslv, c0, c1, c2, stage):
    cid = lax.axis_index("c")
    sid = lax.axis_index("s")
    wid = sid * 2 + cid
    lanes_f = lax.iota(jnp.int32, 16).astype(jnp.float32)
    ones = jnp.full((16,), 1.0, jnp.float32)
    zeros = jnp.full((16,), 0.0, jnp.float32)

    def _vsum(ref, nv):
        def body(i, acc):
            return acc + ref[pl.ds(pl.multiple_of(i * 16, 16), 16)]
        return jnp.sum(lax.fori_loop(0, nv, body, zeros))

    @pl.when(wid < _B)
    def _():
        pltpu.sync_copy(neg_hbm.at[wid], negv)
        pltpu.sync_copy(posl_hbm.at[wid], poslv)
        np_b = _vsum(poslv, _BLK // 16)
        kk = jnp.minimum(3.0 * np_b, float(_N))

        def zero_hist(cref, nv):
            def body(i, _):
                cref[pl.ds(pl.multiple_of(i * 16, 16), 16)] = zeros
                return 0
            lax.fori_loop(0, nv, body, 0)

        def sweep(level, p0, p01):
            def body(i, _):
                for u in range(_SWU):
                    off = pl.ds(pl.multiple_of((i * _SWU + u) * 16, 16), 16)
                    bits = plsc.bitcast(negv[off], jnp.int32)
                    if level == 0:
                        idx = lax.shift_right_logical(bits, 20)
                        plsc.addupdate_scatter(c0, [idx], ones)
                    elif level == 1:
                        msk = lax.shift_right_logical(bits, 20) == p0
                        idx = lax.shift_right_logical(bits, 10) & 1023
                        plsc.addupdate_scatter(c1, [idx], ones, mask=msk)
                    else:
                        msk = lax.shift_right_logical(bits, 10) == p01
                        idx = bits & 1023
                        plsc.addupdate_scatter(c2, [idx], ones, mask=msk)
                return 0
            lax.fori_loop(0, _SCV // _SWU, body, 0)

        def scan(cref, nv, k_lvl):
            def body(t, carry):
                cnt_hi, jstar, cab = carry
                i = nv - 1 - t
                c = cref[pl.ds(pl.multiple_of(i * 16, 16), 16)]
                pc = plsc.cumsum(c)
                tcs = jnp.sum(c)
                above = cnt_hi + (tcs - pc)
                sel = ((above < k_lvl) & (above + c >= k_lvl)).astype(jnp.float32)
                jstar = jstar + jnp.sum(sel * (i.astype(jnp.float32) * 16.0 + lanes_f))
                cab = cab + jnp.sum(sel * above)
                return cnt_hi + tcs, jstar, cab
            return lax.fori_loop(0, nv, body, (0.0, 0.0, 0.0))

        zero_hist(c0, 2048 // 16)
        zero_hist(c1, 1024 // 16)
        zero_hist(c2, 1024 // 16)

        sweep(0, 0, 0)
        _, j0, cab0 = scan(c0, 2048 // 16, kk)
        j0i = j0.astype(jnp.int32)

        sweep(1, j0i, 0)
        _, j1, cab1 = scan(c1, 1024 // 16, kk - cab0)
        j1i = j1.astype(jnp.int32)
        p01 = (j0i << 10) | j1i

        sweep(2, 0, p01)
        _, j2, _ = scan(c2, 1024 // 16, kk - cab0 - cab1)
        j2i = j2.astype(jnp.int32)

        tbits = (p01 << 10) | j2i
        tsv = plsc.bitcast(jnp.full((16,), 1, jnp.int32) * tbits, jnp.float32)
        tstar = jnp.max(tsv)
        def gt_body(i, carry):
            cacc, sacc = carry
            for u in range(_SWU):
                off = pl.ds(pl.multiple_of((i * _SWU + u) * 16, 16), 16)
                v = negv[off]
                m = v > tsv
                cacc = cacc + jnp.where(m, 1.0, 0.0)
                sacc = sacc + jnp.where(m, v, 0.0)
            return cacc, sacc
        cacc, sacc = lax.fori_loop(0, _SCV // _SWU, gt_body, (zeros, zeros))
        cab = jnp.sum(cacc)
        sab = jnp.sum(sacc)
        hard_b = jnp.where(kk > 0.0, sab + (kk - cab) * tstar, 0.0)
        stage[...] = jnp.where(lanes_f == 0.0, hard_b,
                               jnp.where(lanes_f == 1.0, np_b, 0.0))
        pltpu.sync_copy(stage, out1_hbm.at[wid])

    @pl.when(wid == _B)
    def _():
        pltpu.sync_copy(locl_hbm.at[0], poslv)
        def body(i, acc):
            return acc + poslv[pl.ds(pl.multiple_of(i * 16, 16), 16)]
        stage[...] = lax.fori_loop(0, _BLK // 16, body, zeros)
        pltpu.sync_copy(stage, out2_hbm.at[0])

    @pl.when(wid == _B + 1)
    def _():
        pltpu.sync_copy(cepl_hbm.at[0], poslv)
        def body(i, acc):
            return acc + poslv[pl.ds(pl.multiple_of(i * 16, 16), 16)]
        stage[...] = lax.fori_loop(0, _BLK // 16, body, zeros)
        pltpu.sync_copy(stage, out2_hbm.at[1])


def _sc_mine(*args):
    return pl.kernel(
        _sc_mine_kernel,
        mesh=plsc.VectorSubcoreMesh(core_axis_name="c", subcore_axis_name="s"),
        compiler_params=pltpu.CompilerParams(needs_layout_passes=False),
        out_type=[
            jax.ShapeDtypeStruct((_B, 16), jnp.float32),
            jax.ShapeDtypeStruct((2, 16), jnp.float32),
        ],
        scratch_types=[
            pltpu.VMEM((_NP,), jnp.float32),
            pltpu.VMEM((_BLK,), jnp.float32),
            pltpu.VMEM((2048,), jnp.float32),
            pltpu.VMEM((1024,), jnp.float32),
            pltpu.VMEM((1024,), jnp.float32),
            pltpu.VMEM((16,), jnp.float32),
        ],
    )(*args)


def kernel(preds_loc_delta, preds_conf, anchors, gt_boxes, gt_labels):
    anchors_xyxy = jnp.concatenate(
        [anchors[:, :2], anchors[:, :2] + anchors[:, 2:]], axis=1)
    gt_xyxy = jnp.concatenate(
        [gt_boxes[..., :2], gt_boxes[..., :2] + gt_boxes[..., 2:]], axis=-1)
    anct = jnp.zeros((4, _NP), jnp.float32).at[:, :_N].set(anchors_xyxy.T)
    gl = jnp.concatenate(
        [gt_xyxy.transpose(0, 2, 1),
         gt_labels.astype(jnp.float32)[:, None, :]], axis=1)
    conf_t = jnp.zeros((_B, _NUM_CLASSES, _NP), jnp.float32).at[:, :, :_N].set(
        preds_conf.transpose(0, 2, 1))
    pld_t = jnp.zeros((_B, 4, _NP), jnp.float32).at[:, :, :_N].set(
        preds_loc_delta.transpose(0, 2, 1))

    neg, posl, locl, cepl = pl.pallas_call(
        _main_kernel,
        grid=(_B, 2, _NB),
        in_specs=[
            pl.BlockSpec((4, _BLK), lambda b, p, j: (0, j)),
            pl.BlockSpec((1, _G, 4), lambda b, p, j: (b, 0, 0)),
            pl.BlockSpec((1, 5, _G), lambda b, p, j: (b, 0, 0)),
            pl.BlockSpec((1, _NUM_CLASSES, _BLK),
                         lambda b, p, j: (b, 0, jnp.where(p == 0, 0, j))),
            pl.BlockSpec((1, 4, _BLK),
                         lambda b, p, j: (b, 0, jnp.where(p == 0, 0, j))),
        ],
        out_specs=[
            pl.BlockSpec((1, 1, 1, _BLK), lambda b, p, j: (b, j, 0, 0)),
            pl.BlockSpec((1, 1, _BLK), lambda b, p, j: (b, 0, 0)),
            pl.BlockSpec((1, 1, _BLK), lambda b, p, j: (0, 0, 0)),
            pl.BlockSpec((1, 1, _BLK), lambda b, p, j: (0, 0, 0)),
        ],
        out_shape=[
            jax.ShapeDtypeStruct((_B, _NB, 1, _BLK), jnp.float32),
            jax.ShapeDtypeStruct((_B, 1, _BLK), jnp.float32),
            jax.ShapeDtypeStruct((1, 1, _BLK), jnp.float32),
            jax.ShapeDtypeStruct((1, 1, _BLK), jnp.float32),
        ],
        scratch_shapes=[
            pltpu.VMEM((_NB, _G, _BLK), jnp.float32),
            pltpu.VMEM((_G, 1), jnp.float32),
        ],
    )(anct, gt_xyxy, gl, conf_t, pld_t)

    out1, out2 = _sc_mine(neg.reshape(_B, _NP), posl.reshape(_B, _BLK),
                          locl.reshape(1, _BLK), cepl.reshape(1, _BLK))
    hard_tot = out1[:, 0].sum()
    np_tot = out1[:, 1].sum()
    locsum = out2[0].sum()
    cepsum = out2[1].sum()
    lloc = locsum / jnp.maximum(np_tot * 4.0, 1.0)
    lconf = (hard_tot + cepsum) / jnp.maximum(np_tot, 1.0)
    return lloc, lconf


# speedup vs baseline: 1.0529x; 1.0005x over previous
import jax
import jax.numpy as jnp
from jax import lax
from jax.experimental import pallas as pl
from jax.experimental.pallas import tpu as pltpu
from jax.experimental.pallas import tpu_sc as plsc

_NUM_CLASSES = 21
_HIGH_T = 0.9
_LOW_T = 0.3
_B, _N, _G = 16, 20000, 32
_BLK = 4096
_NP = 20480
_NB = _NP // _BLK


def _iou_block(anct_ref, gt_ref):
    ax1 = anct_ref[0:1, :]
    ay1 = anct_ref[1:2, :]
    ax2 = anct_ref[2:3, :]
    ay2 = anct_ref[3:4, :]
    g = gt_ref[0]
    gx1 = g[:, 0:1]
    gy1 = g[:, 1:2]
    gx2 = g[:, 2:3]
    gy2 = g[:, 3:4]
    area_g = (gx2 - gx1) * (gy2 - gy1)
    area_a = (ax2 - ax1) * (ay2 - ay1)
    wx = jnp.maximum(jnp.minimum(gx2, ax2) - jnp.maximum(gx1, ax1), 0.0)
    wy = jnp.maximum(jnp.minimum(gy2, ay2) - jnp.maximum(gy1, ay1), 0.0)
    inter = wx * wy
    return inter / ((area_g + area_a) - inter)


def _main_kernel(anct_ref, gt_ref, gl_ref, conf_ref, pld_ref,
                 neg_ref, posl_ref, locl_ref, cepl_ref,
                 mqs_ref, hpg_ref):
    b = pl.program_id(0)
    p = pl.program_id(1)
    j = pl.program_id(2)

    @pl.when(p == 0)
    def _():
        mq = _iou_block(anct_ref, gt_ref)
        mqs_ref[pl.ds(j, 1)] = mq[None]
        part = jnp.max(mq, axis=1, keepdims=True)

        @pl.when(j == 0)
        def _():
            hpg_ref[...] = part

        @pl.when(j > 0)
        def _():
            hpg_ref[...] = jnp.maximum(hpg_ref[...], part)

    @pl.when(p == 1)
    def _():
        _match_and_losses(anct_ref, gt_ref, gl_ref, conf_ref, pld_ref,
                          neg_ref, posl_ref, locl_ref, cepl_ref,
                          mqs_ref, hpg_ref, b, j)


def _match_and_losses(anct_ref, gt_ref, gl_ref, conf_ref, pld_ref,
                      neg_ref, posl_ref, locl_ref, cepl_ref,
                      mqs_ref, hpg_ref, b, j):
    mq = mqs_ref[pl.ds(j, 1)][0]
    mv = jnp.max(mq, axis=0, keepdims=True)
    giota = jax.lax.broadcasted_iota(jnp.int32, mq.shape, 0).astype(jnp.float32)
    am = jnp.min(jnp.where(mq == mv, giota, float(_G)), axis=0, keepdims=True)
    m = jnp.where(mv < _LOW_T, -1.0, am)
    m = jnp.where((mv >= _LOW_T) & (mv < _HIGH_T), -2.0, m)
    eq = (mq == hpg_ref[...]).astype(jnp.float32)
    restore = jnp.max(eq, axis=0, keepdims=True) > 0.0
    mi = jnp.where(restore, am, m)
    lane = jax.lax.broadcasted_iota(jnp.int32, (1, _BLK), 1)
    pad = (j * _BLK + lane) >= _N
    mi = jnp.where(pad, -1.0, mi)

    idx = jnp.maximum(mi, 0.0)
    soh = (giota == idx).astype(jnp.float32)
    m5 = jnp.dot(gl_ref[0], soh, preferred_element_type=jnp.float32)
    labm = m5[4:5, :]
    ml = jnp.where(mi < 0.0, 0.0, labm)
    pos = ml > 0.0
    posf = pos.astype(jnp.float32)

    ax1 = anct_ref[0:1, :]
    ay1 = anct_ref[1:2, :]
    ax2 = anct_ref[2:3, :]
    ay2 = anct_ref[3:4, :]
    aw = ax2 - ax1
    ah = ay2 - ay1
    acx = (ax1 + ax2) * 0.5
    acy = (ay1 + ay2) * 0.5
    mx1 = m5[0:1, :]
    my1 = m5[1:2, :]
    mx2 = m5[2:3, :]
    my2 = m5[3:4, :]
    mw = mx2 - mx1
    mh = my2 - my1
    mcx = (mx1 + mx2) * 0.5
    mcy = (my1 + my2) * 0.5
    gcx = (mcx - acx) / (0.1 * aw)
    gcy = (mcy - acy) / (0.1 * ah)
    gw = jnp.log(mw / aw) / 0.2
    gh = jnp.log(mh / ah) / 0.2

    pld = pld_ref[0]

    def _sl1(d):
        ad = jnp.abs(d)
        return jnp.where(ad < 1.0, 0.5 * d * d, ad - 0.5)

    lrow = (_sl1(pld[0:1, :] - gcx) + _sl1(pld[1:2, :] - gcy)
            + _sl1(pld[2:3, :] - gw) + _sl1(pld[3:4, :] - gh)) * posf

    x = conf_ref[0]
    e = jnp.exp(x)
    s = jnp.sum(e, axis=0, keepdims=True)
    cio = jax.lax.broadcasted_iota(jnp.int32, x.shape, 0).astype(jnp.float32)
    ohc = (cio == ml).astype(jnp.float32)
    xl = jnp.sum(x * ohc, axis=0, keepdims=True)
    ce = jnp.log(s) - xl

    @pl.when(j == 0)
    def _():
        posl_ref[0] = posf

    @pl.when(j > 0)
    def _():
        posl_ref[0] += posf

    @pl.when((b == 0) & (j == 0))
    def _():
        locl_ref[0] = lrow
        cepl_ref[0] = ce * posf

    @pl.when((b > 0) | (j > 0))
    def _():
        locl_ref[0] += lrow
        cepl_ref[0] += ce * posf

    neg_ref[0, 0] = jnp.where(pos | pad, 0.0, ce)


_SCV = _NP // 16
_SWU = 16


def _sc_mine_kernel(neg_hbm, posl_hbm, locl_hbm, cepl_hbm,
                    out1_hbm, out2_hbm,
                    negv, poslv, c0, c1, c2, stage):
    cid = lax.axis_index("c")
    sid = lax.axis_index("s")
    wid = sid * 2 + cid
    lanes_f = lax.iota(jnp.int32, 16).astype(jnp.float32)
    ones = jnp.full((16,), 1.0, jnp.float32)
    zeros = jnp.full((16,), 0.0, jnp.float32)

    def _vsum(ref, nv):
        def body(i, acc):
            return acc + ref[pl.ds(pl.multiple_of(i * 16, 16), 16)]
        return jnp.sum(lax.fori_loop(0, nv, body, zeros))

    @pl.when(wid < _B)
    def _():
        pltpu.sync_copy(neg_hbm.at[wid], negv)
        pltpu.sync_copy(posl_hbm.at[wid], poslv)
        np_b = _vsum(poslv, _BLK // 16)
        kk = jnp.minimum(3.0 * np_b, float(_N))

        def zero_hist(cref, nv):
            def body(i, _):
                cref[pl.ds(pl.multiple_of(i * 16, 16), 16)] = zeros
                return 0
            lax.fori_loop(0, nv, body, 0)

        def sweep(level, p0, p01):
            def body(i, _):
                for u in range(_SWU):
                    off = pl.ds(pl.multiple_of((i * _SWU + u) * 16, 16), 16)
                    bits = plsc.bitcast(negv[off], jnp.int32)
                    if level == 0:
                        idx = lax.shift_right_logical(bits, 20)
                        plsc.addupdate_scatter(c0, [idx], ones)
                    elif level == 1:
                        msk = lax.shift_right_logical(bits, 20) == p0
                        idx = lax.shift_right_logical(bits, 10) & 1023
                        plsc.addupdate_scatter(c1, [idx], ones, mask=msk)
                    else:
                        msk = lax.shift_right_logical(bits, 10) == p01
                        idx = bits & 1023
                        plsc.addupdate_scatter(c2, [idx], ones, mask=msk)
                return 0
            lax.fori_loop(0, _SCV // _SWU, body, 0)

        def scan(cref, nv, k_lvl):
            def body(t, carry):
                cnt_hi, jstar, cab = carry
                i = nv - 1 - t
                c = cref[pl.ds(pl.multiple_of(i * 16, 16), 16)]
                pc = plsc.cumsum(c)
                tcs = jnp.sum(c)
                above = cnt_hi + (tcs - pc)
                sel = ((above < k_lvl) & (above + c >= k_lvl)).astype(jnp.float32)
                jstar = jstar + jnp.sum(sel * (i.astype(jnp.float32) * 16.0 + lanes_f))
                cab = cab + jnp.sum(sel * above)
                return cnt_hi + tcs, jstar, cab
            return lax.fori_loop(0, nv, body, (0.0, 0.0, 0.0))

        zero_hist(c0, 2048 // 16)
        zero_hist(c1, 1024 // 16)
        zero_hist(c2, 1024 // 16)

        sweep(0, 0, 0)
        _, j0, cab0 = scan(c0, 2048 // 16, kk)
        j0i = j0.astype(jnp.int32)

        sweep(1, j0i, 0)
        _, j1, cab1 = scan(c1, 1024 // 16, kk - cab0)
        j1i = j1.astype(jnp.int32)
        p01 = (j0i << 10) | j1i

        sweep(2, 0, p01)
        _, j2, _ = scan(c2, 1024 // 16, kk - cab0 - cab1)
        j2i = j2.astype(jnp.int32)

        tbits = (p01 << 10) | j2i
        tsv = plsc.bitcast(jnp.full((16,), 1, jnp.int32) * tbits, jnp.float32)
        tstar = jnp.max(tsv)
        def gt_body(i, carry):
            cacc, sacc = carry
            for u in range(_SWU):
                off = pl.ds(pl.multiple_of((i * _SWU + u) * 16, 16), 16)
                v = negv[off]
                m = v > tsv
                cacc = cacc + jnp.where(m, 1.0, 0.0)
                sacc = sacc + jnp.where(m, v, 0.0)
            return cacc, sacc
        cacc, sacc = lax.fori_loop(0, _SCV // _SWU, gt_body, (zeros, zeros))
        cab = jnp.sum(cacc)
        sab = jnp.sum(sacc)
        hard_b = jnp.where(kk > 0.0, sab + (kk - cab) * tstar, 0.0)
        stage[...] = jnp.where(lanes_f == 0.0, hard_b,
                               jnp.where(lanes_f == 1.0, np_b, 0.0))
        pltpu.sync_copy(stage, out1_hbm.at[wid])

    @pl.when(wid == _B)
    def _():
        pltpu.sync_copy(locl_hbm.at[0], poslv)
        def body(i, acc):
            return acc + poslv[pl.ds(pl.multiple_of(i * 16, 16), 16)]
        stage[...] = lax.fori_loop(0, _BLK // 16, body, zeros)
        pltpu.sync_copy(stage, out2_hbm.at[0])

    @pl.when(wid == _B + 1)
    def _():
        pltpu.sync_copy(cepl_hbm.at[0], poslv)
        def body(i, acc):
            return acc + poslv[pl.ds(pl.multiple_of(i * 16, 16), 16)]
        stage[...] = lax.fori_loop(0, _BLK // 16, body, zeros)
        pltpu.sync_copy(stage, out2_hbm.at[1])


def _sc_mine(*args):
    return pl.kernel(
        _sc_mine_kernel,
        mesh=plsc.VectorSubcoreMesh(core_axis_name="c", subcore_axis_name="s"),
        compiler_params=pltpu.CompilerParams(needs_layout_passes=False),
        out_type=[
            jax.ShapeDtypeStruct((_B, 16), jnp.float32),
            jax.ShapeDtypeStruct((2, 16), jnp.float32),
        ],
        scratch_types=[
            pltpu.VMEM((_NP,), jnp.float32),
            pltpu.VMEM((_BLK,), jnp.float32),
            pltpu.VMEM((2048,), jnp.float32),
            pltpu.VMEM((1024,), jnp.float32),
            pltpu.VMEM((1024,), jnp.float32),
            pltpu.VMEM((16,), jnp.float32),
        ],
    )(*args)


def kernel(preds_loc_delta, preds_conf, anchors, gt_boxes, gt_labels):
    anchors_xyxy = jnp.concatenate(
        [anchors[:, :2], anchors[:, :2] + anchors[:, 2:]], axis=1)
    gt_xyxy = jnp.concatenate(
        [gt_boxes[..., :2], gt_boxes[..., :2] + gt_boxes[..., 2:]], axis=-1)
    anct = jnp.zeros((4, _NP), jnp.float32).at[:, :_N].set(anchors_xyxy.T)
    gl = jnp.concatenate(
        [gt_xyxy.transpose(0, 2, 1),
         gt_labels.astype(jnp.float32)[:, None, :]], axis=1)
    conf_t = jnp.zeros((_B, _NUM_CLASSES, _NP), jnp.float32).at[:, :, :_N].set(
        preds_conf.transpose(0, 2, 1))
    pld_t = jnp.zeros((_B, 4, _NP), jnp.float32).at[:, :, :_N].set(
        preds_loc_delta.transpose(0, 2, 1))

    neg, posl, locl, cepl = pl.pallas_call(
        _main_kernel,
        grid=(_B, 2, _NB),
        in_specs=[
            pl.BlockSpec((4, _BLK), lambda b, p, j: (0, j)),
            pl.BlockSpec((1, _G, 4), lambda b, p, j: (b, 0, 0)),
            pl.BlockSpec((1, 5, _G), lambda b, p, j: (b, 0, 0)),
            pl.BlockSpec((1, _NUM_CLASSES, _BLK),
                         lambda b, p, j: (b, 0, jnp.where(p == 0, 0, j))),
            pl.BlockSpec((1, 4, _BLK),
                         lambda b, p, j: (b, 0, jnp.where(p == 0, 0, j))),
        ],
        out_specs=[
            pl.BlockSpec((1, 1, 1, _BLK), lambda b, p, j: (b, j, 0, 0)),
            pl.BlockSpec((1, 1, _BLK), lambda b, p, j: (b, 0, 0)),
            pl.BlockSpec((1, 1, _BLK), lambda b, p, j: (0, 0, 0)),
            pl.BlockSpec((1, 1, _BLK), lambda b, p, j: (0, 0, 0)),
        ],
        out_shape=[
            jax.ShapeDtypeStruct((_B, _NB, 1, _BLK), jnp.float32),
            jax.ShapeDtypeStruct((_B, 1, _BLK), jnp.float32),
            jax.ShapeDtypeStruct((1, 1, _BLK), jnp.float32),
            jax.ShapeDtypeStruct((1, 1, _BLK), jnp.float32),
        ],
        scratch_shapes=[
            pltpu.VMEM((_NB, _G, _BLK), jnp.float32),
            pltpu.VMEM((_G, 1), jnp.float32),
        ],
    )(anct, gt_xyxy, gl, conf_t, pld_t)

    out1, out2 = _sc_mine(neg.reshape(_B, _NP), posl.reshape(_B, _BLK),
                          locl.reshape(1, _BLK), cepl.reshape(1, _BLK))
    hard_tot = out1[:, 0].sum()
    np_tot = out1[:, 1].sum()
    locsum = out2[0].sum()
    cepsum = out2[1].sum()
    lloc = locsum / jnp.maximum(np_tot * 4.0, 1.0)
    lconf = (hard_tot + cepsum) / jnp.maximum(np_tot, 1.0)
    return lloc, lconf
